# baseline (speedup 1.0000x reference)
import jax
import jax.numpy as jnp
from jax.experimental import pallas as pl
from jax.experimental.pallas import tpu as pltpu
from jax.experimental.pallas import tpu_sc as plsc

NBINS = 64
MAX_DIST = 2.0
TB = 2048


def _stage1(x_ref, c_ref, t_ref, pos_ref, neg_ref, cm2_ref, c2b_ref):
    C, D = c_ref.shape

    @pl.when(pl.program_id(0) == 0)
    def _():
        cb0 = c_ref[...]
        cm2_ref[...] = -2.0 * cb0
        c2col = jnp.sum(cb0 * cb0, axis=1, keepdims=True)
        c2b_ref[...] = jax.lax.dot_general(
            c2col, jnp.ones((1, TB), jnp.float32),
            (((1,), (0,)), ((), ())),
            preferred_element_type=jnp.float32)

    xb = x_ref[...]
    tb = t_ref[...]
    xct2 = jax.lax.dot_general(
        cm2_ref[...], xb, (((1,), (1,)), ((), ())),
        preferred_element_type=jnp.float32)
    e = xct2 + c2b_ref[...]
    row = jax.lax.broadcasted_iota(jnp.int32, (C, TB), 0)
    eq = row == tb
    pose = jnp.sum(jnp.where(eq, e, 0.0), axis=0, keepdims=True)
    nege = jnp.min(jnp.where(eq, jnp.inf, e), axis=0, keepdims=True)
    x2t = jax.lax.dot_general(
        jnp.ones((1, D), jnp.float32), xb * xb, (((1,), (1,)), ((), ())),
        preferred_element_type=jnp.float32)
    pos_ref[...] = jnp.sqrt(
        jnp.clip(pose + x2t, 1e-12, None)).reshape(1, 1, TB)
    neg_ref[...] = jnp.sqrt(
        jnp.clip(nege + x2t, 1e-12, None)).reshape(1, 1, TB)


_SC_TILES = 16
_B = 16384
_CHUNK = _B // _SC_TILES
_NV = _CHUNK // 16
_L = 16


def _sc_stage2(pos_hbm, neg_hbm, out_hbm,
               pos_v, neg_v, hist_v, cdf_v, flat_v, stat_v, all_v, allh_v,
               outv_v, sh_mx, sh_mn, sh_hist, sh_part):
    wid = jax.lax.axis_index("s")
    base = wid * _CHUNK
    pltpu.sync_copy(pos_hbm.at[pl.ds(base, _CHUNK)], pos_v)
    pltpu.sync_copy(neg_hbm.at[pl.ds(base, _CHUNK)], neg_v)

    mxv = jnp.full((_L,), -jnp.inf, jnp.float32)
    mnv = jnp.full((_L,), jnp.inf, jnp.float32)
    for i in range(_NV):
        hv = pos_v[pl.ds(i * _L, _L)] - neg_v[pl.ds(i * _L, _L)]
        mxv = jnp.maximum(mxv, hv)
        mnv = jnp.minimum(mnv, hv)
    stat_v[...] = mxv
    pltpu.sync_copy(stat_v, sh_mx.at[pl.ds(wid * _L, _L)])
    stat_v[...] = mnv
    pltpu.sync_copy(stat_v, sh_mn.at[pl.ds(wid * _L, _L)])
    plsc.subcore_barrier()

    pltpu.sync_copy(sh_mx, all_v)
    m = all_v[pl.ds(0, _L)]
    for r in range(1, _SC_TILES):
        m = jnp.maximum(m, all_v[pl.ds(r * _L, _L)])
    mx = jnp.maximum(jnp.max(m), MAX_DIST)
    pltpu.sync_copy(sh_mn, all_v)
    m = all_v[pl.ds(0, _L)]
    for r in range(1, _SC_TILES):
        m = jnp.minimum(m, all_v[pl.ds(r * _L, _L)])
    mn_s = jnp.minimum(jnp.min(m), -MAX_DIST)
    mn = jnp.full((_L,), mn_s, jnp.float32)
    bw = (jnp.full((_L,), mx, jnp.float32) - mn) / (NBINS - 1.0)

    lane = jax.lax.broadcasted_iota(jnp.int32, (_L,), 0)
    lane_base = lane * NBINS
    for j in range(_L * NBINS // _L):
        flat_v[pl.ds(j * _L, _L)] = jnp.zeros((_L,), jnp.float32)
    for i in range(_NV):
        p = pos_v[pl.ds(i * _L, _L)]
        n = neg_v[pl.ds(i * _L, _L)]
        hv = p - n
        t = (hv - mn) / bw
        lo = t.astype(jnp.int32)
        lof = lo.astype(jnp.float32)
        alpha = 1.0 - (hv - mn - lof * bw) / bw
        hi = jnp.minimum(lo + 1, NBINS - 1)
        plsc.addupdate_scatter(flat_v, [lane_base + lo], alpha)
        plsc.addupdate_scatter(flat_v, [lane_base + hi], 1.0 - alpha)
    for j in range(NBINS // _L):
        acc = flat_v[pl.ds(j * _L, _L)]
        for l in range(1, _L):
            acc = acc + flat_v[pl.ds(l * NBINS + j * _L, _L)]
        hist_v[pl.ds(j * _L, _L)] = acc
    pltpu.sync_copy(hist_v, sh_hist.at[pl.ds(wid * NBINS, NBINS)])
    plsc.subcore_barrier()

    pltpu.sync_copy(sh_hist, allh_v)
    hs = []
    for j in range(NBINS // _L):
        acc = allh_v[pl.ds(j * _L, _L)]
        for r in range(1, _SC_TILES):
            acc = acc + allh_v[pl.ds(r * NBINS + j * _L, _L)]
        hs.append(acc)
    s = 0.0
    for j in range(NBINS // _L):
        s = s + jnp.sum(hs[j])
    s_v = jnp.full((_L,), s, jnp.float32)
    prev = 0.0
    for j in range(NBINS // _L):
        cj = plsc.cumsum(hs[j]) + prev
        prev = prev + jnp.sum(hs[j])
        cdf_v[pl.ds(j * _L, _L)] = cj / s_v

    accp = jnp.zeros((_L,), jnp.float32)
    accn = jnp.zeros((_L,), jnp.float32)
    for i in range(_NV):
        p = pos_v[pl.ds(i * _L, _L)]
        n = neg_v[pl.ds(i * _L, _L)]
        hv = p - n
        lo = ((hv - mn) / bw).astype(jnp.int32)
        w = plsc.load_gather(cdf_v, [lo])
        accp = accp + p * w
        accn = accn + n * w
    stat_v[...] = accp - accn
    pltpu.sync_copy(stat_v, sh_part.at[pl.ds(wid * _L, _L)])
    plsc.subcore_barrier()

    @pl.when(wid == 0)
    def _():
        pltpu.sync_copy(sh_part, all_v)
        acc = all_v[pl.ds(0, _L)]
        for r in range(1, _SC_TILES):
            acc = acc + all_v[pl.ds(r * _L, _L)]
        loss = jnp.sum(acc) * (1.0 / _B)
        lane = jax.lax.broadcasted_iota(jnp.int32, (_L,), 0)
        outv_v[...] = jnp.where(lane == 0, loss, 0.0)
        pltpu.sync_copy(outv_v, out_hbm)


def _sc_stage2_call(pos_flat, neg_flat):
    mesh = plsc.VectorSubcoreMesh(
        core_axis_name="c", subcore_axis_name="s", num_cores=1)
    f = pl.kernel(
        _sc_stage2,
        out_type=jax.ShapeDtypeStruct((_L,), jnp.float32),
        mesh=mesh,
        scratch_types=[
            pltpu.VMEM((_CHUNK,), jnp.float32),
            pltpu.VMEM((_CHUNK,), jnp.float32),
            pltpu.VMEM((NBINS,), jnp.float32),
            pltpu.VMEM((NBINS,), jnp.float32),
            pltpu.VMEM((_L * NBINS,), jnp.float32),
            pltpu.VMEM((_L,), jnp.float32),
            pltpu.VMEM((_SC_TILES * _L,), jnp.float32),
            pltpu.VMEM((_SC_TILES * NBINS,), jnp.float32),
            pltpu.VMEM((_L,), jnp.float32),
            pltpu.VMEM_SHARED((_SC_TILES * _L,), jnp.float32),
            pltpu.VMEM_SHARED((_SC_TILES * _L,), jnp.float32),
            pltpu.VMEM_SHARED((_SC_TILES * NBINS,), jnp.float32),
            pltpu.VMEM_SHARED((_SC_TILES * _L,), jnp.float32),
        ],
        compiler_params=pltpu.CompilerParams(needs_layout_passes=False),
    )
    return f(pos_flat, neg_flat)


def kernel(x, targets, centers):
    B, D = x.shape
    C = centers.shape[0]
    tgt = targets.astype(jnp.int32).reshape(1, B)
    NB = B // TB

    pos2, neg2 = pl.pallas_call(
        _stage1,
        grid=(NB,),
        in_specs=[
            pl.BlockSpec((TB, D), lambda i: (i, 0)),
            pl.BlockSpec((C, D), lambda i: (0, 0)),
            pl.BlockSpec((1, TB), lambda i: (0, i)),
        ],
        out_specs=[
            pl.BlockSpec((1, 1, TB), lambda i: (i, 0, 0)),
            pl.BlockSpec((1, 1, TB), lambda i: (i, 0, 0)),
        ],
        out_shape=[
            jax.ShapeDtypeStruct((NB, 1, TB), jnp.float32),
            jax.ShapeDtypeStruct((NB, 1, TB), jnp.float32),
        ],
        scratch_shapes=[
            pltpu.VMEM((C, D), jnp.float32),
            pltpu.VMEM((C, TB), jnp.float32),
        ],
    )(x, centers, tgt)

    loss_vec = _sc_stage2_call(pos2.reshape(B), neg2.reshape(B))
    return loss_vec[0].reshape(())


# speedup vs baseline: 1.0614x; 1.0614x over previous
import jax
import jax.numpy as jnp
from jax.experimental import pallas as pl
from jax.experimental.pallas import tpu as pltpu
from jax.experimental.pallas import tpu_sc as plsc

NBINS = 64
MAX_DIST = 2.0
TB = 1024


def _stage1(x_ref, c_ref, t_ref, pos_ref, neg_ref, caug_ref, xaug_ref):
    C, D = c_ref.shape
    K = D + 8

    @pl.when(pl.program_id(0) == 0)
    def _():
        cb0 = c_ref[...]
        caug_ref[:, :D] = -2.0 * cb0
        c2col = jnp.sum(cb0 * cb0, axis=1, keepdims=True)
        ctail = jax.lax.broadcasted_iota(jnp.int32, (C, K - D), 1)
        caug_ref[:, D:] = jnp.where(ctail == 0, c2col, 0.0)
        xtail = jax.lax.broadcasted_iota(jnp.int32, (TB, K - D), 1)
        xaug_ref[:, D:] = jnp.where(xtail == 0, 1.0, 0.0)

    xb = x_ref[...]
    tb = t_ref[...]
    xaug_ref[:, :D] = xb
    e = jax.lax.dot_general(
        caug_ref[...], xaug_ref[...], (((1,), (1,)), ((), ())),
        preferred_element_type=jnp.float32)
    row = jax.lax.broadcasted_iota(jnp.int32, (C, TB), 0)
    eq = row == tb
    pose = jnp.sum(jnp.where(eq, e, 0.0), axis=0, keepdims=True)
    nege = jnp.min(jnp.where(eq, jnp.inf, e), axis=0, keepdims=True)
    x2t = jax.lax.dot_general(
        jnp.ones((1, D), jnp.float32), xb * xb, (((1,), (1,)), ((), ())),
        preferred_element_type=jnp.float32)
    pos_ref[...] = jnp.sqrt(
        jnp.clip(pose + x2t, 1e-12, None)).reshape(1, 1, TB)
    neg_ref[...] = jnp.sqrt(
        jnp.clip(nege + x2t, 1e-12, None)).reshape(1, 1, TB)


_SC_TILES = 16
_B = 16384
_CHUNK = _B // _SC_TILES
_NV = _CHUNK // 16
_L = 16


def _sc_stage2(pos_hbm, neg_hbm, out_hbm,
               pos_v, neg_v, hist_v, cdf_v, flat_v, stat_v, all_v, allh_v,
               outv_v, sh_mx, sh_mn, sh_hist, sh_part):
    wid = jax.lax.axis_index("s")
    base = wid * _CHUNK
    pltpu.sync_copy(pos_hbm.at[pl.ds(base, _CHUNK)], pos_v)
    pltpu.sync_copy(neg_hbm.at[pl.ds(base, _CHUNK)], neg_v)

    mxv = jnp.full((_L,), -jnp.inf, jnp.float32)
    mnv = jnp.full((_L,), jnp.inf, jnp.float32)
    for i in range(_NV):
        hv = pos_v[pl.ds(i * _L, _L)] - neg_v[pl.ds(i * _L, _L)]
        mxv = jnp.maximum(mxv, hv)
        mnv = jnp.minimum(mnv, hv)
    stat_v[...] = mxv
    pltpu.sync_copy(stat_v, sh_mx.at[pl.ds(wid * _L, _L)])
    stat_v[...] = mnv
    pltpu.sync_copy(stat_v, sh_mn.at[pl.ds(wid * _L, _L)])
    plsc.subcore_barrier()

    pltpu.sync_copy(sh_mx, all_v)
    m = all_v[pl.ds(0, _L)]
    for r in range(1, _SC_TILES):
        m = jnp.maximum(m, all_v[pl.ds(r * _L, _L)])
    mx = jnp.maximum(jnp.max(m), MAX_DIST)
    pltpu.sync_copy(sh_mn, all_v)
    m = all_v[pl.ds(0, _L)]
    for r in range(1, _SC_TILES):
        m = jnp.minimum(m, all_v[pl.ds(r * _L, _L)])
    mn_s = jnp.minimum(jnp.min(m), -MAX_DIST)
    mn = jnp.full((_L,), mn_s, jnp.float32)
    bw = (jnp.full((_L,), mx, jnp.float32) - mn) / (NBINS - 1.0)

    lane = jax.lax.broadcasted_iota(jnp.int32, (_L,), 0)
    lane_base = lane * NBINS
    for j in range(_L * NBINS // _L):
        flat_v[pl.ds(j * _L, _L)] = jnp.zeros((_L,), jnp.float32)
    for i in range(_NV):
        p = pos_v[pl.ds(i * _L, _L)]
        n = neg_v[pl.ds(i * _L, _L)]
        hv = p - n
        t = (hv - mn) / bw
        lo = t.astype(jnp.int32)
        lof = lo.astype(jnp.float32)
        alpha = 1.0 - (hv - mn - lof * bw) / bw
        hi = jnp.minimum(lo + 1, NBINS - 1)
        plsc.addupdate_scatter(flat_v, [lane_base + lo], alpha)
        plsc.addupdate_scatter(flat_v, [lane_base + hi], 1.0 - alpha)
    for j in range(NBINS // _L):
        acc = flat_v[pl.ds(j * _L, _L)]
        for l in range(1, _L):
            acc = acc + flat_v[pl.ds(l * NBINS + j * _L, _L)]
        hist_v[pl.ds(j * _L, _L)] = acc
    pltpu.sync_copy(hist_v, sh_hist.at[pl.ds(wid * NBINS, NBINS)])
    plsc.subcore_barrier()

    pltpu.sync_copy(sh_hist, allh_v)
    hs = []
    for j in range(NBINS // _L):
        acc = allh_v[pl.ds(j * _L, _L)]
        for r in range(1, _SC_TILES):
            acc = acc + allh_v[pl.ds(r * NBINS + j * _L, _L)]
        hs.append(acc)
    s = 0.0
    for j in range(NBINS // _L):
        s = s + jnp.sum(hs[j])
    s_v = jnp.full((_L,), s, jnp.float32)
    prev = 0.0
    for j in range(NBINS // _L):
        cj = plsc.cumsum(hs[j]) + prev
        prev = prev + jnp.sum(hs[j])
        cdf_v[pl.ds(j * _L, _L)] = cj / s_v

    accp = jnp.zeros((_L,), jnp.float32)
    accn = jnp.zeros((_L,), jnp.float32)
    for i in range(_NV):
        p = pos_v[pl.ds(i * _L, _L)]
        n = neg_v[pl.ds(i * _L, _L)]
        hv = p - n
        lo = ((hv - mn) / bw).astype(jnp.int32)
        w = plsc.load_gather(cdf_v, [lo])
        accp = accp + p * w
        accn = accn + n * w
    stat_v[...] = accp - accn
    pltpu.sync_copy(stat_v, sh_part.at[pl.ds(wid * _L, _L)])
    plsc.subcore_barrier()

    @pl.when(wid == 0)
    def _():
        pltpu.sync_copy(sh_part, all_v)
        acc = all_v[pl.ds(0, _L)]
        for r in range(1, _SC_TILES):
            acc = acc + all_v[pl.ds(r * _L, _L)]
        loss = jnp.sum(acc) * (1.0 / _B)
        lane = jax.lax.broadcasted_iota(jnp.int32, (_L,), 0)
        outv_v[...] = jnp.where(lane == 0, loss, 0.0)
        pltpu.sync_copy(outv_v, out_hbm)


def _sc_stage2_call(pos_flat, neg_flat):
    mesh = plsc.VectorSubcoreMesh(
        core_axis_name="c", subcore_axis_name="s", num_cores=1)
    f = pl.kernel(
        _sc_stage2,
        out_type=jax.ShapeDtypeStruct((_L,), jnp.float32),
        mesh=mesh,
        scratch_types=[
            pltpu.VMEM((_CHUNK,), jnp.float32),
            pltpu.VMEM((_CHUNK,), jnp.float32),
            pltpu.VMEM((NBINS,), jnp.float32),
            pltpu.VMEM((NBINS,), jnp.float32),
            pltpu.VMEM((_L * NBINS,), jnp.float32),
            pltpu.VMEM((_L,), jnp.float32),
            pltpu.VMEM((_SC_TILES * _L,), jnp.float32),
            pltpu.VMEM((_SC_TILES * NBINS,), jnp.float32),
            pltpu.VMEM((_L,), jnp.float32),
            pltpu.VMEM_SHARED((_SC_TILES * _L,), jnp.float32),
            pltpu.VMEM_SHARED((_SC_TILES * _L,), jnp.float32),
            pltpu.VMEM_SHARED((_SC_TILES * NBINS,), jnp.float32),
            pltpu.VMEM_SHARED((_SC_TILES * _L,), jnp.float32),
        ],
        compiler_params=pltpu.CompilerParams(needs_layout_passes=False),
    )
    return f(pos_flat, neg_flat)


def kernel(x, targets, centers):
    B, D = x.shape
    C = centers.shape[0]
    tgt = targets.astype(jnp.int32).reshape(1, B)
    NB = B // TB

    pos2, neg2 = pl.pallas_call(
        _stage1,
        grid=(NB,),
        in_specs=[
            pl.BlockSpec((TB, D), lambda i: (i, 0)),
            pl.BlockSpec((C, D), lambda i: (0, 0)),
            pl.BlockSpec((1, TB), lambda i: (0, i)),
        ],
        out_specs=[
            pl.BlockSpec((1, 1, TB), lambda i: (i, 0, 0)),
            pl.BlockSpec((1, 1, TB), lambda i: (i, 0, 0)),
        ],
        out_shape=[
            jax.ShapeDtypeStruct((NB, 1, TB), jnp.float32),
            jax.ShapeDtypeStruct((NB, 1, TB), jnp.float32),
        ],
        scratch_shapes=[
            pltpu.VMEM((C, D + 8), jnp.float32),
            pltpu.VMEM((TB, D + 8), jnp.float32),
        ],
    )(x, centers, tgt)

    loss_vec = _sc_stage2_call(pos2.reshape(B), neg2.reshape(B))
    return loss_vec[0].reshape(())
